# baseline (speedup 1.0000x reference)
import jax
import jax.numpy as jnp
from jax.experimental import pallas as pl
from jax.experimental.pallas import tpu as pltpu

E = 8
TOPK = 2
D = 1024
F = 2048
ALPHA = 0.01

N = 2048
TM = 128
NT = (TOPK * N) // TM + E
PN = NT * TM
LANES = 128


def _router_body(x_ref, wr_ref, logits_ref, meta_ref, stats_ref):
    x = x_ref[...]
    lg = jax.lax.dot_general(
        x, wr_ref[...], (((1,), (0,)), ((), ())),
        preferred_element_type=jnp.float32)
    logits_ref[...] = lg
    col = jax.lax.broadcasted_iota(jnp.int32, (N, LANES), 1)
    neg = jnp.float32(-1e30)
    lm = jnp.where(col < E, lg, neg)
    v1 = jnp.max(lm, axis=1, keepdims=True)
    i1 = jnp.min(jnp.where(lm == v1, col, LANES), axis=1, keepdims=True)
    lm2 = jnp.where(col == i1, neg, lm)
    v2 = jnp.max(lm2, axis=1, keepdims=True)
    i2 = jnp.min(jnp.where(lm2 == v2, col, LANES), axis=1, keepdims=True)
    t = jnp.exp(v2 - v1)
    w_b = t / (1.0 + t)
    w_a = 1.0 - w_b

    p = jnp.where(col < E, jnp.exp(lm - v1), 0.0)
    probs = p / jnp.sum(p, axis=1, keepdims=True)
    psum = jnp.sum(probs, axis=0, keepdims=True)
    oh1 = jnp.where(col == i1, 1.0, 0.0)
    oh2 = jnp.where(col == i2, 1.0, 0.0)
    g1 = jnp.sum(oh1, axis=0, keepdims=True)
    g2 = jnp.sum(oh2, axis=0, keepdims=True)
    g = g1 + g2
    aux = jnp.float32(ALPHA * E) * jnp.sum(psum * g) / jnp.float32(N * N)

    tri = (jax.lax.broadcasted_iota(jnp.int32, (N, N), 0)
           >= jax.lax.broadcasted_iota(jnp.int32, (N, N), 1)
           ).astype(jnp.bfloat16)
    c1 = jax.lax.dot_general(
        tri, oh1.astype(jnp.bfloat16), (((1,), (0,)), ((), ())),
        preferred_element_type=jnp.float32)
    c2 = jax.lax.dot_general(
        tri, oh2.astype(jnp.bfloat16), (((1,), (0,)), ((), ())),
        preferred_element_type=jnp.float32)
    rank1 = jnp.sum(oh1 * c1, axis=1, keepdims=True) - 1.0
    rank2 = (jnp.sum(oh2 * (g1 + c2), axis=1, keepdims=True) - 1.0)
    tiles = jnp.floor((g + jnp.float32(TM - 1)) / jnp.float32(TM))
    triu = (jax.lax.broadcasted_iota(jnp.int32, (LANES, LANES), 0)
            <= jax.lax.broadcasted_iota(jnp.int32, (LANES, LANES), 1)
            ).astype(jnp.bfloat16)
    tile_cum = jax.lax.dot_general(
        tiles.astype(jnp.bfloat16), triu, (((1,), (0,)), ((), ())),
        preferred_element_type=jnp.float32)
    row_off = jnp.float32(TM) * (tile_cum - tiles)
    pos1 = jnp.sum(oh1 * row_off, axis=1, keepdims=True) + rank1
    pos2 = jnp.sum(oh2 * row_off, axis=1, keepdims=True) + rank2

    meta_ref[...] = (
        jnp.where(col == 0, i1.astype(jnp.float32), 0.0)
        + jnp.where(col == 1, i2.astype(jnp.float32), 0.0)
        + jnp.where(col == 2, w_a, 0.0)
        + jnp.where(col == 3, w_b, 0.0)
        + jnp.where(col == 4, pos1, 0.0)
        + jnp.where(col == 5, pos2, 0.0))
    row = jax.lax.broadcasted_iota(jnp.int32, (8, LANES), 0)
    stats_ref[...] = (
        jnp.where(row == 0, aux, 0.0)
        + jnp.where(row == 1, tile_cum, 0.0))


def _ffn_body(te_ref, valid_ref, xs_ref, w1_ref, w3_ref, w2_ref, out_ref):
    i = pl.program_id(0)

    @pl.when(valid_ref[i] == 1)
    def _compute():
        x = xs_ref[...]
        h1 = jax.lax.dot_general(
            x, w1_ref[0].astype(jnp.bfloat16), (((1,), (1,)), ((), ())),
            preferred_element_type=jnp.float32)
        h3 = jax.lax.dot_general(
            x, w3_ref[0].astype(jnp.bfloat16), (((1,), (1,)), ((), ())),
            preferred_element_type=jnp.float32)
        g = (h1 * jax.nn.sigmoid(h1) * h3).astype(jnp.bfloat16)
        out_ref[...] = jax.lax.dot_general(
            g, w2_ref[0].astype(jnp.bfloat16), (((1,), (1,)), ((), ())),
            preferred_element_type=jnp.float32).astype(jnp.bfloat16)

    @pl.when(valid_ref[i] == 0)
    def _zero():
        out_ref[...] = jnp.zeros_like(out_ref)


@jax.jit
def kernel(x, Wr, w1, w3, w2):
    xt = x.reshape(N, D)
    wr_pad = jnp.zeros((D, LANES), jnp.float32).at[:, :E].set(Wr.T)

    logits_pad, meta, stats = pl.pallas_call(
        _router_body,
        out_shape=[
            jax.ShapeDtypeStruct((N, LANES), jnp.float32),
            jax.ShapeDtypeStruct((N, LANES), jnp.float32),
            jax.ShapeDtypeStruct((8, LANES), jnp.float32),
        ],
    )(xt, wr_pad)

    logits = logits_pad[:, :E]
    aux_loss = stats[0, 0]
    w_a = meta[:, 2]
    w_b = meta[:, 3]
    pos1 = meta[:, 4].astype(jnp.int32)
    pos2 = meta[:, 5].astype(jnp.int32)

    tile_cum = stats[1, :E]
    tile_expert = jnp.minimum(
        jnp.searchsorted(tile_cum, jnp.arange(NT, dtype=jnp.float32),
                         side="right"),
        E - 1).astype(jnp.int32)
    tile_valid = (jnp.arange(NT) < tile_cum[E - 1]).astype(jnp.int32)

    tok = jnp.arange(N, dtype=jnp.int32)
    xt_b = xt.astype(jnp.bfloat16)
    row_src = (jnp.zeros((PN,), jnp.int32)
               .at[pos1].set(tok).at[pos2].set(tok))
    xs = xt_b[row_src]

    grid_spec = pltpu.PrefetchScalarGridSpec(
        num_scalar_prefetch=2,
        grid=(NT,),
        in_specs=[
            pl.BlockSpec((TM, D), lambda i, te, va: (i, 0)),
            pl.BlockSpec((1, F, D), lambda i, te, va: (te[i], 0, 0)),
            pl.BlockSpec((1, F, D), lambda i, te, va: (te[i], 0, 0)),
            pl.BlockSpec((1, D, F), lambda i, te, va: (te[i], 0, 0)),
        ],
        out_specs=pl.BlockSpec((TM, D), lambda i, te, va: (i, 0)),
    )
    ys = pl.pallas_call(
        _ffn_body,
        grid_spec=grid_spec,
        out_shape=jax.ShapeDtypeStruct((PN, D), jnp.bfloat16),
    )(tile_expert, tile_valid, xs, w1, w3, w2)

    out = w_a[:, None] * ys[pos1] + w_b[:, None] * ys[pos2]
    return out.reshape(1, N, D), aux_loss, logits.reshape(1, N, E)


# speedup vs baseline: 1.4568x; 1.4568x over previous
import jax
import jax.numpy as jnp
from jax.experimental import pallas as pl
from jax.experimental.pallas import tpu as pltpu

E = 8
TOPK = 2
D = 1024
F = 2048
ALPHA = 0.01

N = 2048
TM = 512
NT = (TOPK * N) // TM + E
PN = NT * TM
LANES = 128


def _router_body(x_ref, wr_ref, logits_ref, meta_ref, stats_ref):
    x = x_ref[...]
    lg = jax.lax.dot_general(
        x, wr_ref[...], (((1,), (0,)), ((), ())),
        preferred_element_type=jnp.float32)
    logits_ref[...] = lg
    col = jax.lax.broadcasted_iota(jnp.int32, (N, LANES), 1)
    neg = jnp.float32(-1e30)
    lm = jnp.where(col < E, lg, neg)
    v1 = jnp.max(lm, axis=1, keepdims=True)
    i1 = jnp.min(jnp.where(lm == v1, col, LANES), axis=1, keepdims=True)
    lm2 = jnp.where(col == i1, neg, lm)
    v2 = jnp.max(lm2, axis=1, keepdims=True)
    i2 = jnp.min(jnp.where(lm2 == v2, col, LANES), axis=1, keepdims=True)
    t = jnp.exp(v2 - v1)
    w_b = t / (1.0 + t)
    w_a = 1.0 - w_b

    p = jnp.where(col < E, jnp.exp(lm - v1), 0.0)
    probs = p / jnp.sum(p, axis=1, keepdims=True)
    psum = jnp.sum(probs, axis=0, keepdims=True)
    oh1 = jnp.where(col == i1, 1.0, 0.0)
    oh2 = jnp.where(col == i2, 1.0, 0.0)
    g1 = jnp.sum(oh1, axis=0, keepdims=True)
    g2 = jnp.sum(oh2, axis=0, keepdims=True)
    g = g1 + g2
    aux = jnp.float32(ALPHA * E) * jnp.sum(psum * g) / jnp.float32(N * N)

    tri = (jax.lax.broadcasted_iota(jnp.int32, (N, N), 0)
           >= jax.lax.broadcasted_iota(jnp.int32, (N, N), 1)
           ).astype(jnp.bfloat16)
    c1 = jax.lax.dot_general(
        tri, oh1.astype(jnp.bfloat16), (((1,), (0,)), ((), ())),
        preferred_element_type=jnp.float32)
    c2 = jax.lax.dot_general(
        tri, oh2.astype(jnp.bfloat16), (((1,), (0,)), ((), ())),
        preferred_element_type=jnp.float32)
    rank1 = jnp.sum(oh1 * c1, axis=1, keepdims=True) - 1.0
    rank2 = (jnp.sum(oh2 * (g1 + c2), axis=1, keepdims=True) - 1.0)
    tiles = jnp.floor((g + jnp.float32(TM - 1)) / jnp.float32(TM))
    triu = (jax.lax.broadcasted_iota(jnp.int32, (LANES, LANES), 0)
            <= jax.lax.broadcasted_iota(jnp.int32, (LANES, LANES), 1)
            ).astype(jnp.bfloat16)
    tile_cum = jax.lax.dot_general(
        tiles.astype(jnp.bfloat16), triu, (((1,), (0,)), ((), ())),
        preferred_element_type=jnp.float32)
    row_off = jnp.float32(TM) * (tile_cum - tiles)
    pos1 = jnp.sum(oh1 * row_off, axis=1, keepdims=True) + rank1
    pos2 = jnp.sum(oh2 * row_off, axis=1, keepdims=True) + rank2

    meta_ref[...] = (
        jnp.where(col == 0, i1.astype(jnp.float32), 0.0)
        + jnp.where(col == 1, i2.astype(jnp.float32), 0.0)
        + jnp.where(col == 2, w_a, 0.0)
        + jnp.where(col == 3, w_b, 0.0)
        + jnp.where(col == 4, pos1, 0.0)
        + jnp.where(col == 5, pos2, 0.0))
    row = jax.lax.broadcasted_iota(jnp.int32, (8, LANES), 0)
    stats_ref[...] = (
        jnp.where(row == 0, aux, 0.0)
        + jnp.where(row == 1, tile_cum, 0.0))


def _ffn_body(te_ref, valid_ref, xs_ref, w1_ref, w3_ref, w2_ref, out_ref):
    i = pl.program_id(0)

    @pl.when(valid_ref[i] == 1)
    def _compute():
        x = xs_ref[...]
        h1 = jax.lax.dot_general(
            x, w1_ref[0].astype(jnp.bfloat16), (((1,), (1,)), ((), ())),
            preferred_element_type=jnp.float32)
        h3 = jax.lax.dot_general(
            x, w3_ref[0].astype(jnp.bfloat16), (((1,), (1,)), ((), ())),
            preferred_element_type=jnp.float32)
        g = (h1 * jax.nn.sigmoid(h1) * h3).astype(jnp.bfloat16)
        out_ref[...] = jax.lax.dot_general(
            g, w2_ref[0].astype(jnp.bfloat16), (((1,), (1,)), ((), ())),
            preferred_element_type=jnp.float32).astype(jnp.bfloat16)

    @pl.when(valid_ref[i] == 0)
    def _zero():
        out_ref[...] = jnp.zeros_like(out_ref)


@jax.jit
def kernel(x, Wr, w1, w3, w2):
    xt = x.reshape(N, D)
    wr_pad = jnp.zeros((D, LANES), jnp.float32).at[:, :E].set(Wr.T)

    logits_pad, meta, stats = pl.pallas_call(
        _router_body,
        out_shape=[
            jax.ShapeDtypeStruct((N, LANES), jnp.float32),
            jax.ShapeDtypeStruct((N, LANES), jnp.float32),
            jax.ShapeDtypeStruct((8, LANES), jnp.float32),
        ],
    )(xt, wr_pad)

    logits = logits_pad[:, :E]
    aux_loss = stats[0, 0]
    w_a = meta[:, 2]
    w_b = meta[:, 3]
    pos1 = meta[:, 4].astype(jnp.int32)
    pos2 = meta[:, 5].astype(jnp.int32)

    tile_cum = stats[1, :E]
    tile_expert = jnp.minimum(
        jnp.searchsorted(tile_cum, jnp.arange(NT, dtype=jnp.float32),
                         side="right"),
        E - 1).astype(jnp.int32)
    tile_valid = (jnp.arange(NT) < tile_cum[E - 1]).astype(jnp.int32)

    tok = jnp.arange(N, dtype=jnp.int32)
    xt_b = xt.astype(jnp.bfloat16)
    row_src = (jnp.zeros((PN,), jnp.int32)
               .at[pos1].set(tok).at[pos2].set(tok))
    xs = xt_b[row_src]

    grid_spec = pltpu.PrefetchScalarGridSpec(
        num_scalar_prefetch=2,
        grid=(NT,),
        in_specs=[
            pl.BlockSpec((TM, D), lambda i, te, va: (i, 0)),
            pl.BlockSpec((1, F, D), lambda i, te, va: (te[i], 0, 0)),
            pl.BlockSpec((1, F, D), lambda i, te, va: (te[i], 0, 0)),
            pl.BlockSpec((1, D, F), lambda i, te, va: (te[i], 0, 0)),
        ],
        out_specs=pl.BlockSpec((TM, D), lambda i, te, va: (i, 0)),
    )
    ys = pl.pallas_call(
        _ffn_body,
        grid_spec=grid_spec,
        out_shape=jax.ShapeDtypeStruct((PN, D), jnp.bfloat16),
    )(tile_expert, tile_valid, xs, w1, w3, w2)

    out = w_a[:, None] * ys[pos1] + w_b[:, None] * ys[pos2]
    return out.reshape(1, N, D), aux_loss, logits.reshape(1, N, E)
